# Initial kernel scaffold; baseline (speedup 1.0000x reference)
import jax
import jax.numpy as jnp
from jax.experimental import pallas as pl


def kernel(x, edge_index, edge_weight, node_anchor_adj, graph_skip_conn, W0, W1, W2):
    raise NotImplementedError("write your pallas kernel here")


# speedup vs baseline: 3.4619x; 3.4619x over previous
import functools

import jax
import jax.numpy as jnp
from jax import lax
from jax.experimental import pallas as pl
from jax.experimental.pallas import tpu as pltpu
from jax.experimental.pallas import tpu_sc as plsc

N = 10000
E = 320000
NANCH = 128
NHID = 256
NCLASS = 64

NC = 2
NS = 16
LANES = 16
ROW_STRIDE = 624
ROW_SPAN = 640
CHUNK = 80

BN = 1000
NB = N // BN

_EPS = 1e-12


def _sc_spmm(table, row, col, w, zeros, *, D, colsplit):
    epw = E // NS if colsplit else E // (NC * NS)
    n_chunks = epw // CHUNK
    mesh = plsc.VectorSubcoreMesh(core_axis_name="c", subcore_axis_name="s")

    def body(table_r, row_r, col_r, w_r, z_r, out_r,
             colv, rowv, wv, buf, acc, sem):
        c = lax.axis_index("c")
        s = lax.axis_index("s")
        rbase = s * ROW_STRIDE
        pltpu.sync_copy(z_r.at[pl.ds(rbase, ROW_SPAN)],
                        acc.at[pl.ds(rbase, ROW_SPAN)])
        plsc.subcore_barrier()

        base = s * epw if colsplit else (c * NS + s) * epw

        def chunk(j, carry):
            b = base + j * CHUNK
            pltpu.sync_copy(col_r.at[pl.ds(b, CHUNK)], colv)
            pltpu.sync_copy(row_r.at[pl.ds(b, CHUNK)], rowv)
            pltpu.sync_copy(w_r.at[pl.ds(b, CHUNK)], wv.at[pl.ds(LANES, CHUNK)])
            if colsplit:
                off = c * N
                for t in range(CHUNK // LANES):
                    sl = pl.ds(t * LANES, LANES)
                    colv[sl] = colv[sl] + off
            pltpu.async_copy(table_r.at[colv], buf, sem).wait()
            for i in range(CHUNK):
                wsplat = plsc.load_gather(
                    wv, [jnp.full((LANES,), LANES + i, jnp.int32)])
                for t in range(D // LANES):
                    sl = pl.ds(t * LANES, LANES)
                    buf[i, sl] = buf[i, sl] * wsplat
            pltpu.sync_copy(buf, acc.at[rowv], add=True)
            return carry

        lax.fori_loop(0, n_chunks, chunk, 0)
        plsc.subcore_barrier()
        pltpu.sync_copy(acc.at[pl.ds(rbase, ROW_SPAN)],
                        out_r.at[c, pl.ds(rbase, ROW_SPAN)])

    return pl.kernel(
        body,
        out_type=jax.ShapeDtypeStruct((2, N, D), jnp.float32),
        mesh=mesh,
        compiler_params=pltpu.CompilerParams(
            needs_layout_passes=False,
            use_tc_tiling_on_sc=False if D < 128 else None,
        ),
        scratch_types=[
            pltpu.VMEM((CHUNK,), jnp.int32),
            pltpu.VMEM((CHUNK,), jnp.int32),
            pltpu.VMEM((LANES + CHUNK,), jnp.float32),
            pltpu.VMEM((CHUNK, D), jnp.float32),
            pltpu.VMEM_SHARED((N, D), jnp.float32),
            pltpu.SemaphoreType.DMA,
        ],
    )(table, row, col, w, zeros)


def _dotT(a, b):
    return lax.dot_general(a, b, (((0,), (0,)), ((), ())),
                           preferred_element_type=jnp.float32)


def _dot(a, b):
    return lax.dot_general(a, b, (((1,), (0,)), ((), ())),
                           preferred_element_type=jnp.float32)


def _reduce0_body(x_r, adj_r, t0_r, cs_r):
    i = pl.program_id(0)

    @pl.when(i == 0)
    def _():
        t0_r[...] = jnp.zeros_like(t0_r)
        cs_r[...] = jnp.zeros_like(cs_r)

    adjb = adj_r[...]
    t0_r[...] += _dotT(adjb, x_r[...])
    cs_r[...] += jnp.sum(adjb, axis=0, keepdims=True)


def _tc_reduce0(x, adj):
    return pl.pallas_call(
        _reduce0_body,
        grid=(NB,),
        in_specs=[
            pl.BlockSpec((BN, NANCH), lambda i: (i, 0)),
            pl.BlockSpec((BN, NANCH), lambda i: (i, 0)),
        ],
        out_specs=[
            pl.BlockSpec((NANCH, NANCH), lambda i: (0, 0)),
            pl.BlockSpec((1, NANCH), lambda i: (0, 0)),
        ],
        out_shape=[
            jax.ShapeDtypeStruct((NANCH, NANCH), jnp.float32),
            jax.ShapeDtypeStruct((1, NANCH), jnp.float32),
        ],
    )(x, adj)


def _q_body(t_r, w_r, q_r):
    q_r[...] = _dot(t_r[...], w_r[...])


def _tc_q(T, W):
    return pl.pallas_call(
        _q_body,
        out_shape=jax.ShapeDtypeStruct((NANCH, W.shape[1]), jnp.float32),
    )(T, W)


def _rw0_body(adj_r, cs_r, y_r, q_r, w_r, s_r,
              first_r, ia_r, nv1_r, t1_r):
    i = pl.program_id(0)
    adjb = adj_r[...]
    invc = 1.0 / jnp.maximum(cs_r[...], _EPS)
    invr = 1.0 / jnp.maximum(
        jnp.sum(adjb, axis=1, keepdims=True), _EPS)
    first = _dot(adjb * invc, q_r[...]) * invr
    agg = y_r[0] + y_r[1]
    ia = _dot(agg, w_r[...])
    s = s_r[0, 0]
    nv1 = jnp.maximum((1.0 - s) * first + s * ia, 0.0)
    first_r[...] = first
    ia_r[...] = ia
    nv1_r[0] = nv1[:, :NANCH]
    nv1_r[1] = nv1[:, NANCH:]

    @pl.when(i == 0)
    def _():
        t1_r[...] = jnp.zeros_like(t1_r)

    t1_r[...] += _dotT(adjb, nv1)


def _tc_rw0(adj, colsum, Y0, Q0, W0, s2d):
    return pl.pallas_call(
        _rw0_body,
        grid=(NB,),
        in_specs=[
            pl.BlockSpec((BN, NANCH), lambda i: (i, 0)),
            pl.BlockSpec((1, NANCH), lambda i: (0, 0)),
            pl.BlockSpec((2, BN, NANCH), lambda i: (0, i, 0)),
            pl.BlockSpec((NANCH, NHID), lambda i: (0, 0)),
            pl.BlockSpec((NANCH, NHID), lambda i: (0, 0)),
            pl.BlockSpec((1, 1), lambda i: (0, 0)),
        ],
        out_specs=[
            pl.BlockSpec((BN, NHID), lambda i: (i, 0)),
            pl.BlockSpec((BN, NHID), lambda i: (i, 0)),
            pl.BlockSpec((2, BN, NANCH), lambda i: (0, i, 0)),
            pl.BlockSpec((NANCH, NHID), lambda i: (0, 0)),
        ],
        out_shape=[
            jax.ShapeDtypeStruct((N, NHID), jnp.float32),
            jax.ShapeDtypeStruct((N, NHID), jnp.float32),
            jax.ShapeDtypeStruct((2, N, NANCH), jnp.float32),
            jax.ShapeDtypeStruct((NANCH, NHID), jnp.float32),
        ],
    )(adj, colsum, Y0, Q0, W0, s2d)


def _rw1_body(adj_r, cs_r, y_r, q_r, w1_r, w2_r, s_r,
              nv2_r, sup_r, t2_r):
    i = pl.program_id(0)
    adjb = adj_r[...]
    invc = 1.0 / jnp.maximum(cs_r[...], _EPS)
    invr = 1.0 / jnp.maximum(
        jnp.sum(adjb, axis=1, keepdims=True), _EPS)
    mid = _dot(adjb * invc, q_r[...]) * invr
    spmm1 = _dot(y_r[0], w1_r[:NANCH, :]) + _dot(y_r[1], w1_r[NANCH:, :])
    s = s_r[0, 0]
    nv2 = jnp.maximum((1.0 - s) * mid + s * spmm1, 0.0)
    nv2_r[...] = nv2
    sup_r[...] = _dot(nv2, w2_r[...])

    @pl.when(i == 0)
    def _():
        t2_r[...] = jnp.zeros_like(t2_r)

    t2_r[...] += _dotT(adjb, nv2)


def _tc_rw1(adj, colsum, Y1, Q1, W1, W2, s2d):
    return pl.pallas_call(
        _rw1_body,
        grid=(NB,),
        in_specs=[
            pl.BlockSpec((BN, NANCH), lambda i: (i, 0)),
            pl.BlockSpec((1, NANCH), lambda i: (0, 0)),
            pl.BlockSpec((2, BN, NANCH), lambda i: (0, i, 0)),
            pl.BlockSpec((NANCH, NHID), lambda i: (0, 0)),
            pl.BlockSpec((NHID, NHID), lambda i: (0, 0)),
            pl.BlockSpec((NHID, NCLASS), lambda i: (0, 0)),
            pl.BlockSpec((1, 1), lambda i: (0, 0)),
        ],
        out_specs=[
            pl.BlockSpec((BN, NHID), lambda i: (i, 0)),
            pl.BlockSpec((BN, NCLASS), lambda i: (i, 0)),
            pl.BlockSpec((NANCH, NHID), lambda i: (0, 0)),
        ],
        out_shape=[
            jax.ShapeDtypeStruct((N, NHID), jnp.float32),
            jax.ShapeDtypeStruct((N, NCLASS), jnp.float32),
            jax.ShapeDtypeStruct((NANCH, NHID), jnp.float32),
        ],
    )(adj, colsum, Y1, Q1, W1, W2, s2d)


def _rw2_body(adj_r, cs_r, y_r, q_r, s_r, out_r):
    adjb = adj_r[...]
    invc = 1.0 / jnp.maximum(cs_r[...], _EPS)
    invr = 1.0 / jnp.maximum(
        jnp.sum(adjb, axis=1, keepdims=True), _EPS)
    cur = _dot(adjb * invc, q_r[...]) * invr
    agg2 = y_r[0] + y_r[1]
    s = s_r[0, 0]
    o = (1.0 - s) * cur + s * agg2
    o = o - jnp.max(o, axis=1, keepdims=True)
    out_r[...] = o - jnp.log(jnp.sum(jnp.exp(o), axis=1, keepdims=True))


def _tc_rw2(adj, colsum, Y2, Q2, s2d):
    return pl.pallas_call(
        _rw2_body,
        grid=(NB,),
        in_specs=[
            pl.BlockSpec((BN, NANCH), lambda i: (i, 0)),
            pl.BlockSpec((1, NANCH), lambda i: (0, 0)),
            pl.BlockSpec((2, BN, NCLASS), lambda i: (0, i, 0)),
            pl.BlockSpec((NANCH, NCLASS), lambda i: (0, 0)),
            pl.BlockSpec((1, 1), lambda i: (0, 0)),
        ],
        out_specs=pl.BlockSpec((BN, NCLASS), lambda i: (i, 0)),
        out_shape=jax.ShapeDtypeStruct((N, NCLASS), jnp.float32),
    )(adj, colsum, Y2, Q2, s2d)


@jax.jit
def kernel(x, edge_index, edge_weight, node_anchor_adj, graph_skip_conn,
           W0, W1, W2):
    row = edge_index[0]
    col = edge_index[1]
    s2d = graph_skip_conn.reshape(1, 1).astype(jnp.float32)
    zeros128 = jnp.zeros((N, NANCH), jnp.float32)
    zeros64 = jnp.zeros((N, NCLASS), jnp.float32)

    T0, colsum = _tc_reduce0(x, node_anchor_adj)
    Q0 = _tc_q(T0, W0)
    Y0 = _sc_spmm(x, row, col, edge_weight, zeros128, D=NANCH,
                  colsplit=False)
    first_vec, init_agg_vec, nv1st, T1 = _tc_rw0(
        node_anchor_adj, colsum, Y0, Q0, W0, s2d)

    Q1 = _tc_q(T1, W1)
    Y1 = _sc_spmm(nv1st.reshape(2 * N, NANCH), row, col, edge_weight,
                  zeros128, D=NANCH, colsplit=True)
    node_vec, support2, T2 = _tc_rw1(
        node_anchor_adj, colsum, Y1, Q1, W1, W2, s2d)

    Q2 = _tc_q(T2, W2)
    Y2 = _sc_spmm(support2, row, col, edge_weight, zeros64, D=NCLASS,
                  colsplit=False)
    output = _tc_rw2(node_anchor_adj, colsum, Y2, Q2, s2d)

    return (first_vec, init_agg_vec, node_vec, output)


# speedup vs baseline: 4.3044x; 1.2434x over previous
import functools

import jax
import jax.numpy as jnp
from jax import lax
from jax.experimental import pallas as pl
from jax.experimental.pallas import tpu as pltpu
from jax.experimental.pallas import tpu_sc as plsc

N = 10000
E = 320000
NANCH = 128
NHID = 256
NCLASS = 64

NC = 2
NS = 16
LANES = 16
ROW_STRIDE = 624
ROW_SPAN = 640
CHUNK = 80

BN = 1000
NB = N // BN

_EPS = 1e-12


def _sc_spmm(table, row, col, w, zeros, *, D, colsplit):
    epw = E // NS if colsplit else E // (NC * NS)
    n_chunks = epw // CHUNK
    pairs = n_chunks // 2
    odd = n_chunks % 2 == 1
    mesh = plsc.VectorSubcoreMesh(core_axis_name="c", subcore_axis_name="s")

    def body(table_r, row_r, col_r, w_r, z_r, out_r,
             colv, rowv, wv, buf, acc, isem, gsem):
        c = lax.axis_index("c")
        s = lax.axis_index("s")
        rbase = s * ROW_STRIDE
        pltpu.sync_copy(z_r.at[pl.ds(rbase, ROW_SPAN)],
                        acc.at[pl.ds(rbase, ROW_SPAN)])
        plsc.subcore_barrier()

        base = s * epw if colsplit else (c * NS + s) * epw
        off = c * N

        def issue_idx(ch, p):
            b = base + ch * CHUNK
            pltpu.async_copy(col_r.at[pl.ds(b, CHUNK)], colv[p], isem[p])
            pltpu.async_copy(row_r.at[pl.ds(b, CHUNK)], rowv[p], isem[p])
            pltpu.async_copy(w_r.at[pl.ds(b, CHUNK)],
                             wv[p].at[pl.ds(LANES, CHUNK)], isem[p])

        def wait_idx(ch, p):
            b = base + ch * CHUNK
            pltpu.make_async_copy(
                col_r.at[pl.ds(b, CHUNK)], colv[p], isem[p]).wait()
            pltpu.make_async_copy(
                row_r.at[pl.ds(b, CHUNK)], rowv[p], isem[p]).wait()
            pltpu.make_async_copy(
                w_r.at[pl.ds(b, CHUNK)],
                wv[p].at[pl.ds(LANES, CHUNK)], isem[p]).wait()

        def issue_gather(p):
            if colsplit:
                for t in range(CHUNK // LANES):
                    sl = pl.ds(t * LANES, LANES)
                    colv[p][sl] = colv[p][sl] + off
            pltpu.async_copy(table_r.at[colv[p]], buf[p], gsem[p])

        def wait_gather(p):
            pltpu.make_async_copy(table_r.at[colv[p]], buf[p], gsem[p]).wait()

        def process(p):
            for i in range(CHUNK):
                wsplat = plsc.load_gather(
                    wv[p], [jnp.full((LANES,), LANES + i, jnp.int32)])
                for t in range(D // LANES):
                    sl = pl.ds(t * LANES, LANES)
                    buf[p][i, sl] = buf[p][i, sl] * wsplat
            pltpu.sync_copy(buf[p], acc.at[rowv[p]], add=True)

        issue_idx(0, 0)
        wait_idx(0, 0)
        issue_gather(0)
        issue_idx(1, 1)

        def pair(jj, carry):
            ch0 = 2 * jj
            ch1 = ch0 + 1
            wait_idx(ch1, 1)
            issue_gather(1)
            wait_gather(0)
            process(0)

            @pl.when(ch0 + 2 < n_chunks)
            def _():
                issue_idx(ch0 + 2, 0)

            wait_gather(1)
            process(1)

            @pl.when(ch1 + 2 < n_chunks)
            def _():
                issue_idx(ch1 + 2, 1)

            @pl.when(ch0 + 2 < n_chunks)
            def _():
                wait_idx(ch0 + 2, 0)
                issue_gather(0)

            return carry

        lax.fori_loop(0, pairs, pair, 0)
        if odd:
            wait_gather(0)
            process(0)

        plsc.subcore_barrier()
        pltpu.sync_copy(acc.at[pl.ds(rbase, ROW_SPAN)],
                        out_r.at[c, pl.ds(rbase, ROW_SPAN)])

    return pl.kernel(
        body,
        out_type=jax.ShapeDtypeStruct((2, N, D), jnp.float32),
        mesh=mesh,
        compiler_params=pltpu.CompilerParams(
            needs_layout_passes=False,
            use_tc_tiling_on_sc=False if D < 128 else None,
        ),
        scratch_types=[
            [pltpu.VMEM((CHUNK,), jnp.int32)] * 2,
            [pltpu.VMEM((CHUNK,), jnp.int32)] * 2,
            [pltpu.VMEM((LANES + CHUNK,), jnp.float32)] * 2,
            [pltpu.VMEM((CHUNK, D), jnp.float32)] * 2,
            pltpu.VMEM_SHARED((N, D), jnp.float32),
            [pltpu.SemaphoreType.DMA] * 2,
            [pltpu.SemaphoreType.DMA] * 2,
        ],
    )(table, row, col, w, zeros)


def _dotT(a, b):
    return lax.dot_general(a, b, (((0,), (0,)), ((), ())),
                           preferred_element_type=jnp.float32)


def _dot(a, b):
    return lax.dot_general(a, b, (((1,), (0,)), ((), ())),
                           preferred_element_type=jnp.float32)


def _reduce0_body(x_r, adj_r, t0_r, cs_r):
    i = pl.program_id(0)

    @pl.when(i == 0)
    def _():
        t0_r[...] = jnp.zeros_like(t0_r)
        cs_r[...] = jnp.zeros_like(cs_r)

    adjb = adj_r[...]
    t0_r[...] += _dotT(adjb, x_r[...])
    cs_r[...] += jnp.sum(adjb, axis=0, keepdims=True)


def _tc_reduce0(x, adj):
    return pl.pallas_call(
        _reduce0_body,
        grid=(NB,),
        in_specs=[
            pl.BlockSpec((BN, NANCH), lambda i: (i, 0)),
            pl.BlockSpec((BN, NANCH), lambda i: (i, 0)),
        ],
        out_specs=[
            pl.BlockSpec((NANCH, NANCH), lambda i: (0, 0)),
            pl.BlockSpec((1, NANCH), lambda i: (0, 0)),
        ],
        out_shape=[
            jax.ShapeDtypeStruct((NANCH, NANCH), jnp.float32),
            jax.ShapeDtypeStruct((1, NANCH), jnp.float32),
        ],
    )(x, adj)


def _q_body(t_r, w_r, q_r):
    q_r[...] = _dot(t_r[...], w_r[...])


def _tc_q(T, W):
    return pl.pallas_call(
        _q_body,
        out_shape=jax.ShapeDtypeStruct((NANCH, W.shape[1]), jnp.float32),
    )(T, W)


def _rw0_body(adj_r, cs_r, y_r, q_r, w_r, s_r,
              first_r, ia_r, nv1_r, t1_r):
    i = pl.program_id(0)
    adjb = adj_r[...]
    invc = 1.0 / jnp.maximum(cs_r[...], _EPS)
    invr = 1.0 / jnp.maximum(
        jnp.sum(adjb, axis=1, keepdims=True), _EPS)
    first = _dot(adjb * invc, q_r[...]) * invr
    agg = y_r[0] + y_r[1]
    ia = _dot(agg, w_r[...])
    s = s_r[0, 0]
    nv1 = jnp.maximum((1.0 - s) * first + s * ia, 0.0)
    first_r[...] = first
    ia_r[...] = ia
    nv1_r[0] = nv1[:, :NANCH]
    nv1_r[1] = nv1[:, NANCH:]

    @pl.when(i == 0)
    def _():
        t1_r[...] = jnp.zeros_like(t1_r)

    t1_r[...] += _dotT(adjb, nv1)


def _tc_rw0(adj, colsum, Y0, Q0, W0, s2d):
    return pl.pallas_call(
        _rw0_body,
        grid=(NB,),
        in_specs=[
            pl.BlockSpec((BN, NANCH), lambda i: (i, 0)),
            pl.BlockSpec((1, NANCH), lambda i: (0, 0)),
            pl.BlockSpec((2, BN, NANCH), lambda i: (0, i, 0)),
            pl.BlockSpec((NANCH, NHID), lambda i: (0, 0)),
            pl.BlockSpec((NANCH, NHID), lambda i: (0, 0)),
            pl.BlockSpec((1, 1), lambda i: (0, 0)),
        ],
        out_specs=[
            pl.BlockSpec((BN, NHID), lambda i: (i, 0)),
            pl.BlockSpec((BN, NHID), lambda i: (i, 0)),
            pl.BlockSpec((2, BN, NANCH), lambda i: (0, i, 0)),
            pl.BlockSpec((NANCH, NHID), lambda i: (0, 0)),
        ],
        out_shape=[
            jax.ShapeDtypeStruct((N, NHID), jnp.float32),
            jax.ShapeDtypeStruct((N, NHID), jnp.float32),
            jax.ShapeDtypeStruct((2, N, NANCH), jnp.float32),
            jax.ShapeDtypeStruct((NANCH, NHID), jnp.float32),
        ],
    )(adj, colsum, Y0, Q0, W0, s2d)


def _rw1_body(adj_r, cs_r, y_r, q_r, w1_r, w2_r, s_r,
              nv2_r, sup_r, t2_r):
    i = pl.program_id(0)
    adjb = adj_r[...]
    invc = 1.0 / jnp.maximum(cs_r[...], _EPS)
    invr = 1.0 / jnp.maximum(
        jnp.sum(adjb, axis=1, keepdims=True), _EPS)
    mid = _dot(adjb * invc, q_r[...]) * invr
    spmm1 = _dot(y_r[0], w1_r[:NANCH, :]) + _dot(y_r[1], w1_r[NANCH:, :])
    s = s_r[0, 0]
    nv2 = jnp.maximum((1.0 - s) * mid + s * spmm1, 0.0)
    nv2_r[...] = nv2
    sup_r[...] = _dot(nv2, w2_r[...])

    @pl.when(i == 0)
    def _():
        t2_r[...] = jnp.zeros_like(t2_r)

    t2_r[...] += _dotT(adjb, nv2)


def _tc_rw1(adj, colsum, Y1, Q1, W1, W2, s2d):
    return pl.pallas_call(
        _rw1_body,
        grid=(NB,),
        in_specs=[
            pl.BlockSpec((BN, NANCH), lambda i: (i, 0)),
            pl.BlockSpec((1, NANCH), lambda i: (0, 0)),
            pl.BlockSpec((2, BN, NANCH), lambda i: (0, i, 0)),
            pl.BlockSpec((NANCH, NHID), lambda i: (0, 0)),
            pl.BlockSpec((NHID, NHID), lambda i: (0, 0)),
            pl.BlockSpec((NHID, NCLASS), lambda i: (0, 0)),
            pl.BlockSpec((1, 1), lambda i: (0, 0)),
        ],
        out_specs=[
            pl.BlockSpec((BN, NHID), lambda i: (i, 0)),
            pl.BlockSpec((BN, NCLASS), lambda i: (i, 0)),
            pl.BlockSpec((NANCH, NHID), lambda i: (0, 0)),
        ],
        out_shape=[
            jax.ShapeDtypeStruct((N, NHID), jnp.float32),
            jax.ShapeDtypeStruct((N, NCLASS), jnp.float32),
            jax.ShapeDtypeStruct((NANCH, NHID), jnp.float32),
        ],
    )(adj, colsum, Y1, Q1, W1, W2, s2d)


def _rw2_body(adj_r, cs_r, y_r, q_r, s_r, out_r):
    adjb = adj_r[...]
    invc = 1.0 / jnp.maximum(cs_r[...], _EPS)
    invr = 1.0 / jnp.maximum(
        jnp.sum(adjb, axis=1, keepdims=True), _EPS)
    cur = _dot(adjb * invc, q_r[...]) * invr
    agg2 = y_r[0] + y_r[1]
    s = s_r[0, 0]
    o = (1.0 - s) * cur + s * agg2
    o = o - jnp.max(o, axis=1, keepdims=True)
    out_r[...] = o - jnp.log(jnp.sum(jnp.exp(o), axis=1, keepdims=True))


def _tc_rw2(adj, colsum, Y2, Q2, s2d):
    return pl.pallas_call(
        _rw2_body,
        grid=(NB,),
        in_specs=[
            pl.BlockSpec((BN, NANCH), lambda i: (i, 0)),
            pl.BlockSpec((1, NANCH), lambda i: (0, 0)),
            pl.BlockSpec((2, BN, NCLASS), lambda i: (0, i, 0)),
            pl.BlockSpec((NANCH, NCLASS), lambda i: (0, 0)),
            pl.BlockSpec((1, 1), lambda i: (0, 0)),
        ],
        out_specs=pl.BlockSpec((BN, NCLASS), lambda i: (i, 0)),
        out_shape=jax.ShapeDtypeStruct((N, NCLASS), jnp.float32),
    )(adj, colsum, Y2, Q2, s2d)


@jax.jit
def kernel(x, edge_index, edge_weight, node_anchor_adj, graph_skip_conn,
           W0, W1, W2):
    row = edge_index[0]
    col = edge_index[1]
    s2d = graph_skip_conn.reshape(1, 1).astype(jnp.float32)
    zeros128 = jnp.zeros((N, NANCH), jnp.float32)
    zeros64 = jnp.zeros((N, NCLASS), jnp.float32)

    T0, colsum = _tc_reduce0(x, node_anchor_adj)
    Q0 = _tc_q(T0, W0)
    Y0 = _sc_spmm(x, row, col, edge_weight, zeros128, D=NANCH,
                  colsplit=False)
    first_vec, init_agg_vec, nv1st, T1 = _tc_rw0(
        node_anchor_adj, colsum, Y0, Q0, W0, s2d)

    Q1 = _tc_q(T1, W1)
    Y1 = _sc_spmm(nv1st.reshape(2 * N, NANCH), row, col, edge_weight,
                  zeros128, D=NANCH, colsplit=True)
    node_vec, support2, T2 = _tc_rw1(
        node_anchor_adj, colsum, Y1, Q1, W1, W2, s2d)

    Q2 = _tc_q(T2, W2)
    Y2 = _sc_spmm(support2, row, col, edge_weight, zeros64, D=NCLASS,
                  colsplit=False)
    output = _tc_rw2(node_anchor_adj, colsum, Y2, Q2, s2d)

    return (first_vec, init_agg_vec, node_vec, output)
